# baseline (speedup 1.0000x reference)
import functools

import jax
import jax.numpy as jnp
from jax import lax
from jax.experimental import pallas as pl
from jax.experimental.pallas import tpu as pltpu
from jax.experimental.pallas import tpu_sc as plsc

N = 10000
E = 320000
G = 16
D = 128
NC = 2
NS = 16
C = 128
PR = 80
EP = NC * NS * PR * C
NP = N + 16
NR = 10112

_mesh = plsc.VectorSubcoreMesh(core_axis_name="c", subcore_axis_name="s",
                               num_cores=NC, num_subcores=NS)


def _worker():
    cid = lax.axis_index("c")
    sid = lax.axis_index("s")
    w = cid * NS + sid
    return cid, sid, w


def _zero_fill(ref, nwords):
    def body(t, carry):
        ref[pl.ds(t * 16, 16)] = jnp.zeros((16,), jnp.float32)
        return carry
    lax.fori_loop(0, nwords // 16, body, 0)


@functools.partial(
    pl.kernel,
    out_type=jax.ShapeDtypeStruct((NC, NS, NR), jnp.float32),
    mesh=_mesh,
    compiler_params=pltpu.CompilerParams(needs_layout_passes=False),
    scratch_types=[
        pltpu.VMEM((PR * C,), jnp.int32),
        pltpu.VMEM((NR,), jnp.float32),
    ],
)
def _deg_kernel(dst1d, out, dstbuf, hist):
    cid, sid, w = _worker()

    _zero_fill(hist, NR)
    pltpu.sync_copy(dst1d.at[pl.ds(w * PR * C, PR * C)], dstbuf)

    ones16 = jnp.ones((16,), jnp.float32)

    def sc(t, carry):
        for u in range(4):
            d = dstbuf[pl.ds(t * 64 + u * 16, 16)]
            plsc.addupdate_scatter(hist, [d], ones16)
        return carry
    lax.fori_loop(0, PR * C // 64, sc, 0)

    pltpu.sync_copy(hist, out.at[cid, sid])


@functools.partial(
    pl.kernel,
    out_type=jax.ShapeDtypeStruct((NC, NS, NR), jnp.float32),
    mesh=_mesh,
    compiler_params=pltpu.CompilerParams(needs_layout_passes=False),
    scratch_types=[
        pltpu.VMEM((PR * C,), jnp.int32),
        pltpu.VMEM((PR * C,), jnp.int32),
        pltpu.VMEM((PR * C,), jnp.int32),
        pltpu.VMEM((PR * C,), jnp.float32),
        pltpu.VMEM((PR * C,), jnp.int32),
        pltpu.VMEM((NR,), jnp.float32),
        pltpu.VMEM_SHARED((G * NR,), jnp.float32),
        pltpu.VMEM_SHARED((NP,), jnp.int32),
        pltpu.VMEM((NP,), jnp.int32),
        pltpu.SemaphoreType.DMA,
        pltpu.SemaphoreType.DMA,
    ],
)
def _t_kernel(src1d, dst1d, pk, out,
              srcbuf, dstbuf, gbuf, vbuf, ibuf, zbuf, t_sh, pk_sh, pk_vb,
              sem_g, sem_s):
    cid, sid, w = _worker()

    @pl.when(sid == 0)
    def _():
        pltpu.sync_copy(pk, pk_vb)
        pltpu.sync_copy(pk_vb, pk_sh)

    _zero_fill(zbuf, NR)
    pltpu.sync_copy(zbuf, t_sh.at[pl.ds(sid * NR, NR)])

    pltpu.sync_copy(src1d.at[pl.ds(w * PR * C, PR * C)], srcbuf)
    pltpu.sync_copy(dst1d.at[pl.ds(w * PR * C, PR * C)], dstbuf)

    plsc.subcore_barrier()
    pltpu.async_copy(pk_sh.at[dstbuf], gbuf, sem_g).wait()

    def ib(t, carry):
        for q in range(4):
            sl = pl.ds(t * 64 + q * 16, 16)
            u = gbuf[sl]
            g = lax.shift_right_logical(u, 28)
            vbits = lax.shift_left(u & jnp.int32(0x0FFFFFFF), 2)
            vbuf[sl] = plsc.bitcast(vbits, jnp.float32)
            ibuf[sl] = g * NR + srcbuf[sl]
        return carry
    lax.fori_loop(0, PR * C // 64, ib, 0)

    plsc.subcore_barrier()

    pltpu.async_copy(vbuf, t_sh.at[ibuf], sem_s, add=True).wait()

    plsc.subcore_barrier()

    pltpu.sync_copy(t_sh.at[pl.ds(sid * NR, NR)], out.at[cid, sid])


def _dinv_body(dp_ref, ii_ref, dv_ref, pk_ref):
    dsum = jnp.sum(dp_ref[...], axis=0, keepdims=True)[:, :N]
    dinv = lax.rsqrt(dsum + 1.0)
    dv_ref[...] = dinv
    dbits = lax.bitcast_convert_type(dinv, jnp.int32)
    packed = (lax.shift_left(ii_ref[...], 28)
              | lax.shift_right_logical(dbits + 2, 2))
    pk_ref[0, :N] = packed[0]
    pk_ref[0, N:] = jnp.zeros((NP - N,), jnp.int32)


def _head_body(tp_ref, dv_ref, ii_ref, x_ref, w_ref, b_ref, wd_ref, bd_ref,
               o_ref):
    tt = tp_ref[0, :, :N] + tp_ref[1, :, :N]
    dv = dv_ref[...]
    gi = lax.broadcasted_iota(jnp.int32, (G, N), 0)
    mask = ii_ref[...] == gi
    a = dv * tt + jnp.where(mask, dv * dv, 0.0)
    cnt = jnp.sum(mask.astype(jnp.float32), axis=1, keepdims=True)
    h = jnp.dot(x_ref[...], w_ref[...], preferred_element_type=jnp.float32)
    conv = (jnp.dot(a, h, preferred_element_type=jnp.float32,
                    precision=lax.Precision.HIGHEST)
            + cnt * b_ref[...])
    logits = (jnp.dot(conv, wd_ref[...], preferred_element_type=jnp.float32)
              + bd_ref[...])
    m = jnp.max(logits, axis=1, keepdims=True)
    e = jnp.exp(logits - m)
    o_ref[...] = e / jnp.sum(e, axis=1, keepdims=True)


def kernel(x, edge_index, i, W, b, Wd, bd):
    src = edge_index[0].astype(jnp.int32)
    dst = edge_index[1].astype(jnp.int32)
    ii = i.astype(jnp.int32)
    npad = EP - E
    src1d = jnp.concatenate([src, jnp.zeros((npad,), jnp.int32)])
    dst1d = jnp.concatenate([dst, jnp.full((npad,), N, jnp.int32)])

    degp = _deg_kernel(dst1d)
    dinv, pk = pl.pallas_call(
        _dinv_body,
        out_shape=(jax.ShapeDtypeStruct((1, N), jnp.float32),
                   jax.ShapeDtypeStruct((1, NP), jnp.int32)),
    )(degp.reshape(NC * NS, NR), ii.reshape(1, N))
    tp = _t_kernel(src1d, dst1d, pk.reshape(NP))
    out = pl.pallas_call(
        _head_body,
        out_shape=jax.ShapeDtypeStruct((G, 10), jnp.float32),
    )(tp.reshape(NC, G, NR), dinv, ii.reshape(1, N),
      x, W, b.reshape(1, D), Wd, bd.reshape(1, 10))
    return out


# speedup vs baseline: 1.0528x; 1.0528x over previous
import functools

import jax
import jax.numpy as jnp
from jax import lax
from jax.experimental import pallas as pl
from jax.experimental.pallas import tpu as pltpu
from jax.experimental.pallas import tpu_sc as plsc

N = 10000
E = 320000
G = 16
D = 128
NC = 2
NS = 16
C = 128
PR = 80
EP = NC * NS * PR * C
NP = N + 16
NR = 10112

_mesh = plsc.VectorSubcoreMesh(core_axis_name="c", subcore_axis_name="s",
                               num_cores=NC, num_subcores=NS)


def _worker():
    cid = lax.axis_index("c")
    sid = lax.axis_index("s")
    w = cid * NS + sid
    return cid, sid, w


def _zero_fill(ref, nwords):
    z = jnp.zeros((16,), jnp.float32)

    def body(t, carry):
        for u in range(4):
            ref[pl.ds(t * 64 + u * 16, 16)] = z
        return carry
    lax.fori_loop(0, nwords // 64, body, 0)
    for r in range(nwords // 64 * 64, nwords, 16):
        ref[pl.ds(r, 16)] = z


@functools.partial(
    pl.kernel,
    out_type=jax.ShapeDtypeStruct((NC, NS, NR), jnp.float32),
    mesh=_mesh,
    compiler_params=pltpu.CompilerParams(needs_layout_passes=False),
    scratch_types=[
        pltpu.VMEM((PR * C,), jnp.int32),
        pltpu.VMEM((NR,), jnp.float32),
    ],
)
def _deg_kernel(dst1d, out, dstbuf, hist):
    cid, sid, w = _worker()

    _zero_fill(hist, NR)
    pltpu.sync_copy(dst1d.at[pl.ds(w * PR * C, PR * C)], dstbuf)

    ones16 = jnp.ones((16,), jnp.float32)

    def sc(t, carry):
        for u in range(4):
            d = dstbuf[pl.ds(t * 64 + u * 16, 16)]
            plsc.addupdate_scatter(hist, [d], ones16)
        return carry
    lax.fori_loop(0, PR * C // 64, sc, 0)

    pltpu.sync_copy(hist, out.at[cid, sid])


@functools.partial(
    pl.kernel,
    out_type=jax.ShapeDtypeStruct((NC, NS, NR), jnp.float32),
    mesh=_mesh,
    compiler_params=pltpu.CompilerParams(needs_layout_passes=False),
    scratch_types=[
        pltpu.VMEM((PR * C,), jnp.int32),
        pltpu.VMEM((PR * C,), jnp.int32),
        pltpu.VMEM((PR * C,), jnp.int32),
        pltpu.VMEM((PR * C,), jnp.float32),
        pltpu.VMEM((PR * C,), jnp.int32),
        pltpu.VMEM((NR,), jnp.float32),
        pltpu.VMEM_SHARED((G * NR,), jnp.float32),
        pltpu.VMEM_SHARED((NP,), jnp.int32),
        pltpu.VMEM((NP,), jnp.int32),
        pltpu.SemaphoreType.DMA,
        pltpu.SemaphoreType.DMA,
    ],
)
def _t_kernel(src1d, dst1d, pk, out,
              srcbuf, dstbuf, gbuf, vbuf, ibuf, zbuf, t_sh, pk_sh, pk_vb,
              sem_g, sem_s):
    cid, sid, w = _worker()

    @pl.when(sid == 0)
    def _():
        pltpu.sync_copy(pk, pk_vb)
        pltpu.sync_copy(pk_vb, pk_sh)

    _zero_fill(zbuf, NR)
    pltpu.sync_copy(zbuf, t_sh.at[pl.ds(sid * NR, NR)])

    pltpu.sync_copy(src1d.at[pl.ds(w * PR * C, PR * C)], srcbuf)
    pltpu.sync_copy(dst1d.at[pl.ds(w * PR * C, PR * C)], dstbuf)

    plsc.subcore_barrier()
    pltpu.async_copy(pk_sh.at[dstbuf], gbuf, sem_g).wait()

    def ib(t, carry):
        for q in range(4):
            sl = pl.ds(t * 64 + q * 16, 16)
            u = gbuf[sl]
            g = lax.shift_right_logical(u, 28)
            vbits = lax.shift_left(u & jnp.int32(0x0FFFFFFF), 2)
            vbuf[sl] = plsc.bitcast(vbits, jnp.float32)
            ibuf[sl] = g * NR + srcbuf[sl]
        return carry
    lax.fori_loop(0, PR * C // 64, ib, 0)

    plsc.subcore_barrier()

    pltpu.async_copy(vbuf, t_sh.at[ibuf], sem_s, add=True).wait()

    plsc.subcore_barrier()

    pltpu.sync_copy(t_sh.at[pl.ds(sid * NR, NR)], out.at[cid, sid])


def _dinv_body(dp_ref, ii_ref, dv_ref, pk_ref):
    dsum = jnp.sum(dp_ref[...], axis=0, keepdims=True)[:, :N]
    dinv = lax.rsqrt(dsum + 1.0)
    dv_ref[...] = dinv
    dbits = lax.bitcast_convert_type(dinv, jnp.int32)
    packed = (lax.shift_left(ii_ref[...], 28)
              | lax.shift_right_logical(dbits + 2, 2))
    pk_ref[0, :N] = packed[0]
    pk_ref[0, N:] = jnp.zeros((NP - N,), jnp.int32)


def _head_body(tp_ref, dv_ref, ii_ref, x_ref, w_ref, b_ref, wd_ref, bd_ref,
               o_ref):
    tt = tp_ref[0, :, :N] + tp_ref[1, :, :N]
    dv = dv_ref[...]
    gi = lax.broadcasted_iota(jnp.int32, (G, N), 0)
    mask = ii_ref[...] == gi
    a = dv * tt + jnp.where(mask, dv * dv, 0.0)
    cnt = jnp.sum(mask.astype(jnp.float32), axis=1, keepdims=True)
    h = jnp.dot(x_ref[...], w_ref[...], preferred_element_type=jnp.float32)
    conv = (jnp.dot(a, h, preferred_element_type=jnp.float32,
                    precision=lax.Precision.HIGHEST)
            + cnt * b_ref[...])
    logits = (jnp.dot(conv, wd_ref[...], preferred_element_type=jnp.float32)
              + bd_ref[...])
    m = jnp.max(logits, axis=1, keepdims=True)
    e = jnp.exp(logits - m)
    o_ref[...] = e / jnp.sum(e, axis=1, keepdims=True)


def kernel(x, edge_index, i, W, b, Wd, bd):
    src = edge_index[0].astype(jnp.int32)
    dst = edge_index[1].astype(jnp.int32)
    ii = i.astype(jnp.int32)
    npad = EP - E
    src1d = jnp.concatenate([src, jnp.zeros((npad,), jnp.int32)])
    dst1d = jnp.concatenate([dst, jnp.full((npad,), N, jnp.int32)])

    degp = _deg_kernel(dst1d)
    dinv, pk = pl.pallas_call(
        _dinv_body,
        out_shape=(jax.ShapeDtypeStruct((1, N), jnp.float32),
                   jax.ShapeDtypeStruct((1, NP), jnp.int32)),
    )(degp.reshape(NC * NS, NR), ii.reshape(1, N))
    tp = _t_kernel(src1d, dst1d, pk.reshape(NP))
    out = pl.pallas_call(
        _head_body,
        out_shape=jax.ShapeDtypeStruct((G, 10), jnp.float32),
    )(tp.reshape(NC, G, NR), dinv, ii.reshape(1, N),
      x, W, b.reshape(1, D), Wd, bd.reshape(1, 10))
    return out
